# baseline (speedup 1.0000x reference)
import dataclasses
import functools

import jax
import jax.numpy as jnp
from jax import lax
from jax.experimental import pallas as pl
from jax.experimental.pallas import tpu as pltpu
from jax.experimental.pallas import tpu_sc as plsc

_EPS = 1e-15
_NC = 2
_NS = 16
_L = 16
_CH = 32
_G16 = 16
_WB = 32


def _pre_logmap(x):
    n, d = x.shape
    blk = 1000

    def body(x_ref, o_ref):
        xb = x_ref[...]
        pn = jnp.sqrt(jnp.sum(xb * xb, axis=1, keepdims=True))
        pn = jnp.maximum(pn, _EPS)
        z = jnp.clip(pn, -1.0 + 1e-7, 1.0 - 1e-7)
        at = 0.5 * (jnp.log1p(z) - jnp.log1p(-z))
        o_ref[...] = xb / pn * at

    return pl.pallas_call(
        body,
        grid=(n // blk,),
        in_specs=[pl.BlockSpec((blk, d), lambda i: (i, 0))],
        out_specs=pl.BlockSpec((blk, d), lambda i: (i, 0)),
        out_shape=jax.ShapeDtypeStruct((n, d), jnp.float32),
    )(x)


def _spmm_sc(pk, ng, xt, n, d, maxg):
    n2 = n // 2
    rpt = (n2 // _NS) // 8 * 8
    tail = n2 - _NS * rpt
    wfull = rpt // _WB
    wrem = rpt - wfull * _WB

    mesh = plsc.VectorSubcoreMesh(core_axis_name="c", subcore_axis_name="s")
    cp = pltpu.CompilerParams()
    if "needs_layout_passes" in pltpu.CompilerParams.__dataclass_fields__:
        cp = dataclasses.replace(cp, needs_layout_passes=False)

    hd = d // 2

    @functools.partial(
        pl.kernel,
        out_type=jax.ShapeDtypeStruct((n, 2, hd), jnp.float32),
        mesh=mesh,
        compiler_params=cp,
        scratch_types=[
            pltpu.VMEM_SHARED((n2, 2, hd), jnp.float32),
            pltpu.VMEM((_G16, 4, _CH), jnp.int32),
            pltpu.VMEM((4, _CH, 2, hd), jnp.float32),
            pltpu.SemaphoreType.DMA((4,)),
            pltpu.SemaphoreType.DMA((4,)),
        ],
    )
    def run(pk_hbm, ng_hbm, x_hbm, out_hbm, acc, ibuf, rows, gsem, ssem):
        c = lax.axis_index("c")
        s = lax.axis_index("s")
        base = s * rpt

        pltpu.sync_copy(ng_hbm, ibuf.at[0, 0, pl.ds(0, _L)])
        ngv = ibuf[0, 0, pl.ds(0, _L)]
        lanes = lax.broadcasted_iota(jnp.int32, (_L,), 0)
        ngc = jnp.max(jnp.where(lanes == c, ngv, 0))

        @pl.loop(0, _WB)
        def _(i):
            for h in range(2):
                for j in range(hd // _L):
                    rows[0, i, h, pl.ds(j * _L, _L)] = jnp.zeros(
                        (_L,), jnp.float32)

        for k in range(wfull):
            pltpu.sync_copy(rows.at[0], acc.at[pl.ds(base + k * _WB, _WB)])
        if wrem:
            pltpu.sync_copy(rows.at[0, pl.ds(0, wrem)],
                            acc.at[pl.ds(base + wfull * _WB, wrem)])
        if tail:
            @pl.when(s == _NS - 1)
            def _():
                pltpu.sync_copy(rows.at[0, pl.ds(0, tail)],
                                acc.at[pl.ds(_NS * rpt, tail)])
        plsc.subcore_barrier()

        def issue_gather(cc, b):
            pltpu.async_copy(x_hbm.at[ibuf.at[cc, 0]], rows.at[b],
                             gsem.at[b])

        def wait_gather(cc, b):
            pltpu.make_async_copy(x_hbm.at[ibuf.at[cc, 0]], rows.at[b],
                                  gsem.at[b]).wait()

        def wait_scatter(cc, b):
            pltpu.make_async_copy(rows.at[b], acc.at[ibuf.at[cc, 1]],
                                  ssem.at[b]).wait()

        @pl.loop(0, maxg)
        def _(g):
            @pl.when(g < ngc)
            def _():
                @pl.when(g > 0)
                def _():
                    for b in range(4):
                        wait_scatter(_G16 - 4 + b, b)

                pltpu.sync_copy(pk_hbm.at[c, s, g], ibuf)
                issue_gather(0, 0)
                issue_gather(1, 1)
                for cc in range(_G16):
                    b = cc % 4
                    if cc < _G16 - 2:
                        if cc >= 2:
                            wait_scatter(cc - 2, (cc + 2) % 4)
                        issue_gather(cc + 2, (cc + 2) % 4)
                    wait_gather(cc, b)

                    @pl.loop(0, _CH, step=2)
                    def _(e):
                        for u in range(2):
                            wv = plsc.bitcast(
                                plsc.load_gather(
                                    ibuf.at[cc, 2],
                                    [jnp.full((_L,), e + u, jnp.int32)]),
                                jnp.float32)
                            for h in range(2):
                                for j in range(hd // _L):
                                    sl = pl.ds(j * _L, _L)
                                    rows[b, e + u, h, sl] = (
                                        rows[b, e + u, h, sl] * wv)

                    pltpu.async_copy(rows.at[b], acc.at[ibuf.at[cc, 1]],
                                     ssem.at[b], add=True)

        @pl.when(ngc > 0)
        def _():
            for b in range(4):
                wait_scatter(_G16 - 4 + b, b)

        plsc.subcore_barrier()

        obase = c * n2 + base
        for k in range(wfull):
            pltpu.sync_copy(acc.at[pl.ds(base + k * _WB, _WB)],
                            out_hbm.at[pl.ds(obase + k * _WB, _WB)])
        if wrem:
            pltpu.sync_copy(acc.at[pl.ds(base + wfull * _WB, wrem)],
                            out_hbm.at[pl.ds(obase + wfull * _WB, wrem)])
        if tail:
            @pl.when(s == _NS - 1)
            def _():
                pltpu.sync_copy(acc.at[pl.ds(_NS * rpt, tail)],
                                out_hbm.at[pl.ds(c * n2 + _NS * rpt, tail)])

    return run(pk, ng, xt.reshape(n, 2, hd)).reshape(n, d)


def _post(y1, y2, y3, n, d):
    blk = 1000
    g = n // blk

    def body(y1_ref, y2_ref, y3_ref, o_ref, acc):
        p = pl.program_id(0)
        i = pl.program_id(1)
        sb = y1_ref[...] + y2_ref[...] + y3_ref[...]

        @pl.when(jnp.logical_and(p == 0, i == 0))
        def _():
            acc[...] = jnp.zeros_like(acc)

        @pl.when(p == 0)
        def _():
            acc[...] += jnp.sum(sb, axis=0, keepdims=True)

        @pl.when(p == 1)
        def _():
            u = sb - acc[...] / n
            n2v = jnp.sum(u * u, axis=1, keepdims=True)
            un = jnp.maximum(jnp.sqrt(n2v), _EPS)
            f = jnp.tanh(un) / un
            e0 = f * u
            en2 = jnp.sum(e0 * e0, axis=1, keepdims=True)
            en = jnp.maximum(jnp.sqrt(en2), _EPS)
            maxnorm = 1.0 - 4e-3
            scale = jnp.where(en > maxnorm, maxnorm / en, 1.0)
            o_ref[...] = e0 * scale

    return pl.pallas_call(
        body,
        grid=(2, g),
        in_specs=[pl.BlockSpec((blk, d), lambda p, i: (i, 0))] * 3,
        out_specs=pl.BlockSpec((blk, d), lambda p, i: (i, 0)),
        out_shape=jax.ShapeDtypeStruct((n, d), jnp.float32),
        scratch_shapes=[pltpu.VMEM((1, d), jnp.float32)],
    )(y1, y2, y3)


def kernel(x, edge_index, edge_weight):
    n, d = x.shape
    e = edge_index.shape[1]
    n2 = n // 2
    grp = _NS * _G16 * _CH
    maxg = -(-e // grp)

    dst = edge_index[0].astype(jnp.int32)
    src = edge_index[1].astype(jnp.int32)
    w32 = lax.bitcast_convert_type(edge_weight.astype(jnp.float32), jnp.int32)

    m0 = dst < n2
    c_id = jnp.where(m0, 0, 1).astype(jnp.int32)
    p0 = jnp.cumsum(m0.astype(jnp.int32))
    ar = jnp.arange(e, dtype=jnp.int32)
    rank = jnp.where(m0, p0 - 1, ar - p0 + jnp.int32(0))
    tile = rank % _NS
    sit = rank // _NS
    gid = sit // (_G16 * _CH)
    r2 = sit % (_G16 * _CH)
    cc = r2 // _CH
    lane = r2 % _CH
    slot = (((c_id * _NS + tile) * maxg + gid) * _G16 + cc) * _CH + lane

    nslot = 2 * _NS * maxg * _G16 * _CH
    perm = jnp.full((nslot,), e, jnp.int32).at[slot].set(
        ar, mode="promise_in_bounds", unique_indices=True)

    srcp = jnp.concatenate([src, jnp.zeros((1,), jnp.int32)])[perm]
    dstl = jnp.concatenate([dst - c_id * n2, jnp.zeros((1,), jnp.int32)])[perm]
    wp = jnp.concatenate([w32, jnp.zeros((1,), jnp.int32)])[perm]
    shp = (2, _NS, maxg, _G16, _CH)
    pk = jnp.stack([srcp.reshape(shp), dstl.reshape(shp), wp.reshape(shp),
                    jnp.zeros(shp, jnp.int32)], axis=4)

    cnt0 = p0[-1]
    cnt1 = e - cnt0
    gpt0 = -(-(-(-cnt0 // _NS)) // (_G16 * _CH))
    gpt1 = -(-(-(-cnt1 // _NS)) // (_G16 * _CH))
    ng = jnp.zeros((16,), jnp.int32).at[0].set(gpt0).at[1].set(gpt1)

    xt = _pre_logmap(x)
    y1 = _spmm_sc(pk, ng, xt, n, d, maxg)
    y2 = _spmm_sc(pk, ng, y1, n, d, maxg)
    y3 = _spmm_sc(pk, ng, y2, n, d, maxg)
    return _post(y1, y2, y3, n, d)
